# baseline (speedup 1.0000x reference)
import functools

import jax
import jax.numpy as jnp
from jax import lax
from jax.experimental import pallas as pl
from jax.experimental.pallas import tpu as pltpu
from jax.experimental.pallas import tpu_sc as plsc

F32 = jnp.float32
I32 = jnp.int32

NPAD = 10240
CW = 128
ROWS_PER_SUB = NPAD // 16


def _zero_rows(rows, width):
  z = jnp.zeros((16,), F32)

  def body(r, _):
    for h in range(width // 16):
      rows[r, pl.ds(h * 16, 16)] = z
    return 0

  lax.fori_loop(0, 128, body, 0)


def _zero_vec(buf, n):
  z = jnp.zeros((16,), F32)
  for k in range(n // 16):
    buf[pl.ds(k * 16, 16)] = z


def _make_scatter_kernel(h, e_pad, layer):
  n_per_sub = e_pad // 16
  cps = n_per_sub // CW
  assert cps * CW == n_per_sub and cps % 4 == 0 and cps >= 8
  with_deg = layer == 1

  mesh = plsc.VectorSubcoreMesh(core_axis_name="c", subcore_axis_name="s",
                                num_cores=2, num_subcores=16)
  out_type = [
      jax.ShapeDtypeStruct((NPAD, h), F32),
      jax.ShapeDtypeStruct((NPAD, h), F32),
  ]
  nbuf = 4
  scratch = (
      [pltpu.VMEM_SHARED((NPAD, h), F32)]
      + [pltpu.VMEM_SHARED((NPAD, h), F32)]
      + ([pltpu.VMEM_SHARED((NPAD,), F32)] if with_deg else [])
      + [pltpu.VMEM((2, CW), I32)] * nbuf
      + [pltpu.VMEM((CW, h), F32)] * nbuf
      + [pltpu.VMEM((CW,), F32)]
      + [pltpu.VMEM((CW,), F32)]
      + [pltpu.VMEM((h,), F32)]
      + [pltpu.VMEM((h,), F32)]
      + [pltpu.SemaphoreType.DMA] * (2 * nbuf + 1)
  )
  if with_deg:
    out_type += [
        jax.ShapeDtypeStruct((NPAD,), F32),
        jax.ShapeDtypeStruct((NPAD,), F32),
    ]
  else:
    out_type += [jax.ShapeDtypeStruct((NPAD, h), F32)]

  def body(*refs):
    if with_deg:
      (ta, tb, ea, eb, biasa, biasb, oa, ob, dega, degb,
       acc, tbl, dacc) = refs[:13]
      rest = refs[13:]
      rdega = rdegb = wsum = u2w = None
    else:
      (ta, tb, ea, eb, biasa, biasb, rdega, rdegb, wsum,
       oa, ob, u2w, acc, tbl) = refs[:14]
      rest = refs[14:]
      dacc = dega = degb = None
    ebufs = rest[:4]
    rbufs = rest[4:8]
    fbuf = rest[8]
    dbuf = rest[9]
    bbuf = rest[10]
    wbuf = rest[11]
    gsems = rest[12:16]
    isems = rest[16:20]
    dsem = rest[20]
    r0buf = rbufs[0]
    c = lax.axis_index("c")
    s = lax.axis_index("s")
    r0 = s * ROWS_PER_SUB

    _zero_rows(r0buf, h)
    for k in range(ROWS_PER_SUB // CW):
      pltpu.sync_copy(r0buf, acc.at[pl.ds(r0 + k * CW, CW)])
    if with_deg:
      _zero_vec(fbuf, CW)
      for k in range(ROWS_PER_SUB // CW):
        pltpu.sync_copy(fbuf, dacc.at[pl.ds(r0 + k * CW, CW)])
      one = jnp.ones((16,), F32)
      for k in range(CW // 16):
        fbuf[pl.ds(k * 16, 16)] = one

    def stage(table):
      for k in range(ROWS_PER_SUB // CW):
        blk = r0 + k * CW
        pltpu.sync_copy(table.at[pl.ds(blk, CW)], r0buf)
        pltpu.sync_copy(r0buf, tbl.at[pl.ds(blk, CW)])

    @pl.when(c == 0)
    def _():
      stage(ta)

    @pl.when(c == 1)
    def _():
      stage(tb)

    plsc.subcore_barrier()

    def process(table, edges):
      base = s * cps

      def gather(b):
        pltpu.async_copy(tbl.at[ebufs[b].at[0]], rbufs[b], gsems[b])

      def wait_gather(b):
        pltpu.make_async_copy(tbl.at[ebufs[b].at[0]], rbufs[b],
                              gsems[b]).wait()

      def wait_idx(b):
        pltpu.make_async_copy(edges.at[base], ebufs[b], isems[b]).wait()

      def scatter(b):
        if with_deg:
          pltpu.async_copy(fbuf, dacc.at[ebufs[b].at[1]], dsem, add=True)
        pltpu.sync_copy(rbufs[b], acc.at[ebufs[b].at[1]], add=True)
        if with_deg:
          pltpu.make_async_copy(fbuf, dacc.at[ebufs[b].at[1]], dsem).wait()

      pltpu.sync_copy(edges.at[base], ebufs[0])
      gather(0)
      pltpu.sync_copy(edges.at[base + 1], ebufs[1])
      gather(1)
      pltpu.async_copy(edges.at[base + 2], ebufs[2], isems[2])

      def quad(jj, _):
        j0 = jj * 4
        for b in range(4):
          j = j0 + b
          wait_gather(b)
          scatter(b)
          wait_idx((b + 2) % 4)
          gather((b + 2) % 4)
          pltpu.async_copy(edges.at[base + j + 3], ebufs[(b + 3) % 4],
                           isems[(b + 3) % 4])
        return 0

      lax.fori_loop(0, cps // 4 - 1, quad, 0)

      wait_gather(0); scatter(0)
      wait_idx(2); gather(2)
      pltpu.async_copy(edges.at[base + cps - 1], ebufs[3], isems[3])
      wait_gather(1); scatter(1)
      wait_idx(3); gather(3)
      wait_gather(2); scatter(2)
      wait_gather(3); scatter(3)

    @pl.when(c == 0)
    def _():
      process(ta, ea)

    @pl.when(c == 1)
    def _():
      process(tb, eb)

    plsc.subcore_barrier()

    def writeout(out, bias, deg_out, rdeg_in, with_u2w):
      pltpu.sync_copy(bias, bbuf)
      if with_u2w:
        pltpu.sync_copy(wsum, wbuf)
      bias_ch = [bbuf[pl.ds(cc * 16, 16)] for cc in range(h // 16)]
      w_ch = ([wbuf[pl.ds(cc * 16, 16)] for cc in range(h // 16)]
              if with_u2w else None)
      for k in range(ROWS_PER_SUB // CW):
        blk = r0 + k * CW
        pltpu.sync_copy(acc.at[pl.ds(blk, CW)], r0buf)
        if with_deg:
          pltpu.sync_copy(dacc.at[pl.ds(blk, CW)], dbuf)
          for kk in range(CW // 16):
            d = dbuf[pl.ds(kk * 16, 16)]
            dbuf[pl.ds(kk * 16, 16)] = 1.0 / jnp.maximum(d, 1.0)
          pltpu.sync_copy(dbuf, deg_out.at[pl.ds(blk, CW)])
        else:
          pltpu.sync_copy(rdeg_in.at[pl.ds(blk, CW)], dbuf)

        def rowgrp(g, _):
          rv16 = dbuf[pl.ds(g * 16, 16)]
          for i in range(16):
            r = g * 16 + i
            rv = rv16[i]
            for cc in range(h // 16):
              x = r0buf[r, pl.ds(cc * 16, 16)]
              y = x * rv + bias_ch[cc]
              r0buf[r, pl.ds(cc * 16, 16)] = y
              if with_u2w:
                rbufs[1][r, pl.ds(cc * 16, 16)] = y * w_ch[cc]
          return 0

        lax.fori_loop(0, CW // 16, rowgrp, 0)
        pltpu.sync_copy(r0buf, out.at[pl.ds(blk, CW)])
        if with_u2w:
          pltpu.sync_copy(rbufs[1], u2w.at[pl.ds(blk, CW)])

    @pl.when(c == 0)
    def _():
      writeout(oa, biasa, dega, rdega, False)

    @pl.when(c == 1)
    def _():
      writeout(ob, biasb, degb, rdegb, not with_deg)

  return pl.kernel(body, out_type=out_type, mesh=mesh, scratch_types=scratch,
                   compiler_params=pltpu.CompilerParams(
                       use_tc_tiling_on_sc=False))


def _make_head_kernel(b_link, h):
  n_per_w = b_link // 32
  n_chunks = n_per_w // CW
  assert n_chunks * CW == n_per_w and n_chunks % 2 == 0 and n_chunks >= 4

  mesh = plsc.VectorSubcoreMesh(core_axis_name="c", subcore_axis_name="s",
                                num_cores=2, num_subcores=16)

  def body(u2w, i2, edges, bsum, hout,
           tbl_a, tbl_b, e0, e1, ar0, ar1, br0, br1, hbuf, bsv,
           ga0, ga1, gb0, gb1, isem0, isem1):
    c = lax.axis_index("c")
    s = lax.axis_index("s")
    wid = s * 2 + c
    base = wid * n_chunks
    pltpu.sync_copy(bsum, bsv)
    iota = lax.iota(I32, 16)
    bufs = [(e0, ar0, br0, ga0, gb0, isem0), (e1, ar1, br1, ga1, gb1, isem1)]

    for k in range(ROWS_PER_SUB // CW):
      blk = s * ROWS_PER_SUB + k * CW
      pltpu.sync_copy(u2w.at[pl.ds(blk, CW)], ar0)
      pltpu.sync_copy(ar0, tbl_a.at[pl.ds(blk, CW)])
      pltpu.sync_copy(i2.at[pl.ds(blk, CW)], ar0)
      pltpu.sync_copy(ar0, tbl_b.at[pl.ds(blk, CW)])
    plsc.subcore_barrier()

    def gathers(bf):
      pltpu.async_copy(tbl_a.at[bf[0].at[0]], bf[1], bf[3])
      pltpu.async_copy(tbl_b.at[bf[0].at[1]], bf[2], bf[4])

    def wait_gathers(bf):
      pltpu.make_async_copy(tbl_a.at[bf[0].at[0]], bf[1], bf[3]).wait()
      pltpu.make_async_copy(tbl_b.at[bf[0].at[1]], bf[2], bf[4]).wait()

    def compute(bf, j):
      def block(k, _):
        rvec = iota + k * 16
        hv0 = bsv[pl.ds(0, 16)]
        hv1 = jnp.zeros((16,), F32)
        for cc in range(h // 2):
          cv0 = jnp.full((16,), 2 * cc, I32)
          cv1 = jnp.full((16,), 2 * cc + 1, I32)
          hv0 = hv0 + (plsc.load_gather(bf[1], [rvec, cv0]) *
                       plsc.load_gather(bf[2], [rvec, cv0]))
          hv1 = hv1 + (plsc.load_gather(bf[1], [rvec, cv1]) *
                       plsc.load_gather(bf[2], [rvec, cv1]))
        hbuf[pl.ds(k * 16, 16)] = hv0 + hv1
        return 0

      lax.fori_loop(0, CW // 16, block, 0)
      pltpu.sync_copy(hbuf, hout.at[pl.ds((base + j) * CW, CW)])

    pltpu.sync_copy(edges.at[base], e0)
    gathers(bufs[0])
    pltpu.async_copy(edges.at[base + 1], e1, isem1)

    def pair(jj, _):
      j0 = jj * 2
      for b in range(2):
        cur = bufs[b]
        nxt = bufs[1 - b]
        j = j0 + b
        wait_gathers(cur)
        pltpu.make_async_copy(edges.at[base], nxt[0], nxt[5]).wait()
        gathers(nxt)
        compute(cur, j)
        pltpu.async_copy(edges.at[base + j + 2], cur[0], cur[5])
      return 0

    lax.fori_loop(0, (n_chunks - 2) // 2, pair, 0)

    wait_gathers(bufs[0])
    pltpu.make_async_copy(edges.at[base], e1, isem1).wait()
    gathers(bufs[1])
    compute(bufs[0], n_chunks - 2)
    wait_gathers(bufs[1])
    compute(bufs[1], n_chunks - 1)

  return pl.kernel(
      body,
      out_type=jax.ShapeDtypeStruct((b_link,), F32),
      mesh=mesh,
      scratch_types=[
          pltpu.VMEM_SHARED((NPAD, h), F32),
          pltpu.VMEM_SHARED((NPAD, h), F32),
          pltpu.VMEM((2, CW), I32),
          pltpu.VMEM((2, CW), I32),
          pltpu.VMEM((CW, h), F32),
          pltpu.VMEM((CW, h), F32),
          pltpu.VMEM((CW, h), F32),
          pltpu.VMEM((CW, h), F32),
          pltpu.VMEM((CW,), F32),
          pltpu.VMEM((16,), F32),
          pltpu.SemaphoreType.DMA,
          pltpu.SemaphoreType.DMA,
          pltpu.SemaphoreType.DMA,
          pltpu.SemaphoreType.DMA,
          pltpu.SemaphoreType.DMA,
          pltpu.SemaphoreType.DMA,
      ],
    compiler_params=pltpu.CompilerParams(use_tc_tiling_on_sc=False,
                                           needs_layout_passes=False),
  )


def _dot(a, b):
  return jnp.dot(a, b, preferred_element_type=F32,
                 precision=lax.Precision.HIGHEST)


def _pre1_body(xu, xi, w1ui, w1iu, wp1, b1iu, bp1v, yu, yi, bc1):
  yu[...] = _dot(xu[...], w1ui[...])
  wc = _dot(w1iu[...], wp1[...])
  yi[...] = _dot(xi[...], wc)
  bc1[...] = _dot(b1iu[...], wp1[...]) + bp1v[...]


def _pre2_body(user1, item1, w2ui, w2iu, wp2, b2iu, bp2v, wpostt,
               zu, zi, bc2, wsum):
  zu[...] = _dot(user1[...], w2ui[...])
  zi[...] = _dot(item1[...], _dot(w2iu[...], wp2[...]))
  bc2[...] = _dot(b2iu[...], wp2[...]) + bp2v[...]
  wsum[...] = jnp.sum(wpostt[...], axis=0, keepdims=True)


def kernel(x_user, x_item, edge_index_ui, edge_index_iu, edge_label_index,
           snap, W1_ui, b1_ui, W1_iu, b1_iu, Wp1, bp1, Ws1, bs1, qs1,
           W2_ui, b2_ui, W2_iu, b2_iu, Wp2, bp2, Ws2, bs2, qs2,
           Wpost, bpost):
  n_user, d_in = x_user.shape
  n_item = x_item.shape[0]
  h1 = W1_ui.shape[1]
  h2 = W2_ui.shape[1]
  e = edge_index_ui.shape[1]
  b_link = edge_label_index.shape[1]

  e_pad = ((e + 64 * CW - 1) // (64 * CW)) * (64 * CW)
  npd = NPAD

  xu_p = jnp.pad(x_user, ((0, npd - n_user), (0, 0)))
  xi_p = jnp.pad(x_item, ((0, npd - n_item), (0, 0)))
  fill = (npd - 240) + (jnp.arange(e_pad - e, dtype=I32) % 240)
  def pad_edges(ei):
    src = jnp.concatenate([ei[0].astype(I32), fill]).reshape(-1, 1, CW)
    dst = jnp.concatenate([ei[1].astype(I32), fill]).reshape(-1, 1, CW)
    return jnp.concatenate([src, dst], axis=1)
  eui3 = pad_edges(edge_index_ui)
  eiu3 = pad_edges(edge_index_iu)
  elab3 = jnp.concatenate(
      [edge_label_index[0].astype(I32).reshape(-1, 1, CW),
       edge_label_index[1].astype(I32).reshape(-1, 1, CW)], axis=1)

  grid = 8
  blk = npd // grid
  yu, yi, bc1 = pl.pallas_call(
      _pre1_body,
      grid=(grid,),
      in_specs=[
          pl.BlockSpec((blk, d_in), lambda i: (i, 0)),
          pl.BlockSpec((blk, d_in), lambda i: (i, 0)),
          pl.BlockSpec((d_in, h1), lambda i: (0, 0)),
          pl.BlockSpec((d_in, h1), lambda i: (0, 0)),
          pl.BlockSpec((h1, h1), lambda i: (0, 0)),
          pl.BlockSpec((1, h1), lambda i: (0, 0)),
          pl.BlockSpec((1, h1), lambda i: (0, 0)),
      ],
      out_specs=[
          pl.BlockSpec((blk, h1), lambda i: (i, 0)),
          pl.BlockSpec((blk, h1), lambda i: (i, 0)),
          pl.BlockSpec((1, h1), lambda i: (0, 0)),
      ],
      out_shape=[
          jax.ShapeDtypeStruct((npd, h1), F32),
          jax.ShapeDtypeStruct((npd, h1), F32),
          jax.ShapeDtypeStruct((1, h1), F32),
      ],
  )(xu_p, xi_p, W1_ui, W1_iu, Wp1, b1_iu.reshape(1, h1), bp1.reshape(1, h1))

  k2 = _make_scatter_kernel(h1, e_pad, layer=1)
  item1p, user1p, rdeg_i, rdeg_u = k2(yu, yi, eui3, eiu3,
                                      b1_ui, bc1.reshape(h1))

  zu, zi, bc2, wsum = pl.pallas_call(
      _pre2_body,
      grid=(grid,),
      in_specs=[
          pl.BlockSpec((blk, h1), lambda i: (i, 0)),
          pl.BlockSpec((blk, h1), lambda i: (i, 0)),
          pl.BlockSpec((h1, h2), lambda i: (0, 0)),
          pl.BlockSpec((h1, h2), lambda i: (0, 0)),
          pl.BlockSpec((h2, h2), lambda i: (0, 0)),
          pl.BlockSpec((1, h2), lambda i: (0, 0)),
          pl.BlockSpec((1, h2), lambda i: (0, 0)),
          pl.BlockSpec((2, h2), lambda i: (0, 0)),
      ],
      out_specs=[pl.BlockSpec((blk, h2), lambda i: (i, 0))] * 2
      + [pl.BlockSpec((1, h2), lambda i: (0, 0))] * 2,
      out_shape=[
          jax.ShapeDtypeStruct((npd, h2), F32),
          jax.ShapeDtypeStruct((npd, h2), F32),
          jax.ShapeDtypeStruct((1, h2), F32),
          jax.ShapeDtypeStruct((1, h2), F32),
      ],
  )(user1p, item1p, W2_ui, W2_iu, Wp2, b2_iu.reshape(1, h2),
    bp2.reshape(1, h2), Wpost.T)

  k4 = _make_scatter_kernel(h2, e_pad, layer=2)
  item2p, user2p, u2wp = k4(zu, zi, eui3, eiu3, b2_ui, bc2.reshape(h2),
                            rdeg_i, rdeg_u, wsum.reshape(h2))

  bsum = jnp.broadcast_to(jnp.sum(bpost), (16,)).astype(F32)
  k6 = _make_head_kernel(b_link, h2)
  h = k6(u2wp, item2p, elab3, bsum)

  return (h, user1p[:n_user], item1p[:n_item],
          user2p[:n_user], item2p[:n_item])


# speedup vs baseline: 1.1451x; 1.1451x over previous
import functools

import jax
import jax.numpy as jnp
from jax import lax
from jax.experimental import pallas as pl
from jax.experimental.pallas import tpu as pltpu
from jax.experimental.pallas import tpu_sc as plsc

F32 = jnp.float32
I32 = jnp.int32

NPAD = 10240
CW = 128
ROWS_PER_SUB = NPAD // 16


def _zero_rows(rows, width):
  z = jnp.zeros((16,), F32)

  def body(r, _):
    for h in range(width // 16):
      rows[r, pl.ds(h * 16, 16)] = z
    return 0

  lax.fori_loop(0, 128, body, 0)


def _zero_vec(buf, n):
  z = jnp.zeros((16,), F32)
  for k in range(n // 16):
    buf[pl.ds(k * 16, 16)] = z


def _make_scatter_kernel(h, e_pad, layer):
  n_per_sub = e_pad // 16
  cps = n_per_sub // CW
  assert cps * CW == n_per_sub and cps % 4 == 0 and cps >= 8
  with_deg = layer == 1

  mesh = plsc.VectorSubcoreMesh(core_axis_name="c", subcore_axis_name="s",
                                num_cores=2, num_subcores=16)
  out_type = [
      jax.ShapeDtypeStruct((NPAD, h), F32),
      jax.ShapeDtypeStruct((NPAD, h), F32),
  ]
  nbuf = 4
  scratch = (
      [pltpu.VMEM_SHARED((NPAD, h), F32)]
      + ([pltpu.VMEM_SHARED((NPAD,), F32)] if with_deg else [])
      + [pltpu.VMEM((2, CW), I32)] * nbuf
      + [pltpu.VMEM((CW, h), F32)] * nbuf
      + [pltpu.VMEM((CW,), F32)]
      + [pltpu.VMEM((CW,), F32)]
      + [pltpu.VMEM((h,), F32)]
      + [pltpu.VMEM((h,), F32)]
      + [pltpu.SemaphoreType.DMA] * (2 * nbuf + 1)
  )
  if with_deg:
    out_type += [
        jax.ShapeDtypeStruct((NPAD,), F32),
        jax.ShapeDtypeStruct((NPAD,), F32),
    ]
  else:
    out_type += [jax.ShapeDtypeStruct((NPAD, h), F32)]

  def body(*refs):
    if with_deg:
      (ta, tb, ea, eb, biasa, biasb, oa, ob, dega, degb,
       acc, dacc) = refs[:12]
      rest = refs[12:]
      rdega = rdegb = wsum = u2w = None
    else:
      (ta, tb, ea, eb, biasa, biasb, rdega, rdegb, wsum,
       oa, ob, u2w, acc) = refs[:13]
      rest = refs[13:]
      dacc = dega = degb = None
    ebufs = rest[:4]
    rbufs = rest[4:8]
    fbuf = rest[8]
    dbuf = rest[9]
    bbuf = rest[10]
    wbuf = rest[11]
    gsems = rest[12:16]
    isems = rest[16:20]
    dsem = rest[20]
    r0buf = rbufs[0]
    c = lax.axis_index("c")
    s = lax.axis_index("s")
    r0 = s * ROWS_PER_SUB

    _zero_rows(r0buf, h)
    for k in range(ROWS_PER_SUB // CW):
      pltpu.sync_copy(r0buf, acc.at[pl.ds(r0 + k * CW, CW)])
    if with_deg:
      _zero_vec(fbuf, CW)
      for k in range(ROWS_PER_SUB // CW):
        pltpu.sync_copy(fbuf, dacc.at[pl.ds(r0 + k * CW, CW)])
      one = jnp.ones((16,), F32)
      for k in range(CW // 16):
        fbuf[pl.ds(k * 16, 16)] = one

    plsc.subcore_barrier()

    def process(table, edges):
      base = s * cps

      def gather(b):
        pltpu.async_copy(table.at[ebufs[b].at[0]], rbufs[b], gsems[b])

      def wait_gather(b):
        pltpu.make_async_copy(table.at[ebufs[b].at[0]], rbufs[b],
                              gsems[b]).wait()

      def wait_idx(b):
        pltpu.make_async_copy(edges.at[base], ebufs[b], isems[b]).wait()

      def scatter(b):
        if with_deg:
          pltpu.async_copy(fbuf, dacc.at[ebufs[b].at[1]], dsem, add=True)
        pltpu.sync_copy(rbufs[b], acc.at[ebufs[b].at[1]], add=True)
        if with_deg:
          pltpu.make_async_copy(fbuf, dacc.at[ebufs[b].at[1]], dsem).wait()

      pltpu.sync_copy(edges.at[base], ebufs[0])
      gather(0)
      pltpu.sync_copy(edges.at[base + 1], ebufs[1])
      gather(1)
      pltpu.async_copy(edges.at[base + 2], ebufs[2], isems[2])

      def quad(jj, _):
        j0 = jj * 4
        for b in range(4):
          j = j0 + b
          wait_gather(b)
          scatter(b)
          wait_idx((b + 2) % 4)
          gather((b + 2) % 4)
          pltpu.async_copy(edges.at[base + j + 3], ebufs[(b + 3) % 4],
                           isems[(b + 3) % 4])
        return 0

      lax.fori_loop(0, cps // 4 - 1, quad, 0)

      wait_gather(0); scatter(0)
      wait_idx(2); gather(2)
      pltpu.async_copy(edges.at[base + cps - 1], ebufs[3], isems[3])
      wait_gather(1); scatter(1)
      wait_idx(3); gather(3)
      wait_gather(2); scatter(2)
      wait_gather(3); scatter(3)

    @pl.when(c == 0)
    def _():
      process(ta, ea)

    @pl.when(c == 1)
    def _():
      process(tb, eb)

    plsc.subcore_barrier()

    def writeout(out, bias, deg_out, rdeg_in, with_u2w):
      pltpu.sync_copy(bias, bbuf)
      if with_u2w:
        pltpu.sync_copy(wsum, wbuf)
      bias_ch = [bbuf[pl.ds(cc * 16, 16)] for cc in range(h // 16)]
      w_ch = ([wbuf[pl.ds(cc * 16, 16)] for cc in range(h // 16)]
              if with_u2w else None)
      for k in range(ROWS_PER_SUB // CW):
        blk = r0 + k * CW
        pltpu.sync_copy(acc.at[pl.ds(blk, CW)], r0buf)
        if with_deg:
          pltpu.sync_copy(dacc.at[pl.ds(blk, CW)], dbuf)
          for kk in range(CW // 16):
            d = dbuf[pl.ds(kk * 16, 16)]
            dbuf[pl.ds(kk * 16, 16)] = 1.0 / jnp.maximum(d, 1.0)
          pltpu.sync_copy(dbuf, deg_out.at[pl.ds(blk, CW)])
        else:
          pltpu.sync_copy(rdeg_in.at[pl.ds(blk, CW)], dbuf)

        def rowgrp(g, _):
          rv16 = dbuf[pl.ds(g * 16, 16)]
          for i in range(16):
            r = g * 16 + i
            rv = rv16[i]
            for cc in range(h // 16):
              x = r0buf[r, pl.ds(cc * 16, 16)]
              y = x * rv + bias_ch[cc]
              r0buf[r, pl.ds(cc * 16, 16)] = y
              if with_u2w:
                rbufs[1][r, pl.ds(cc * 16, 16)] = y * w_ch[cc]
          return 0

        lax.fori_loop(0, CW // 16, rowgrp, 0)
        pltpu.sync_copy(r0buf, out.at[pl.ds(blk, CW)])
        if with_u2w:
          pltpu.sync_copy(rbufs[1], u2w.at[pl.ds(blk, CW)])

    @pl.when(c == 0)
    def _():
      writeout(oa, biasa, dega, rdega, False)

    @pl.when(c == 1)
    def _():
      writeout(ob, biasb, degb, rdegb, not with_deg)

  return pl.kernel(body, out_type=out_type, mesh=mesh, scratch_types=scratch,
                   compiler_params=pltpu.CompilerParams(
                       use_tc_tiling_on_sc=False))


def _make_head_kernel(b_link, h):
  n_per_w = b_link // 32
  n_chunks = n_per_w // CW
  assert n_chunks * CW == n_per_w and n_chunks % 4 == 0 and n_chunks >= 8

  mesh = plsc.VectorSubcoreMesh(core_axis_name="c", subcore_axis_name="s",
                                num_cores=2, num_subcores=16)

  def body(*refs):
    (u2w, i2, edges, bsum, hout) = refs[:5]
    rest = refs[5:]
    ebufs = rest[:4]
    abufs = rest[4:8]
    bbufs = rest[8:12]
    hbuf = rest[12]
    bsv = rest[13]
    gasems = rest[14:18]
    gbsems = rest[18:22]
    isems = rest[22:26]
    c = lax.axis_index("c")
    s = lax.axis_index("s")
    wid = s * 2 + c
    base = wid * n_chunks
    pltpu.sync_copy(bsum, bsv)
    iota = lax.iota(I32, 16)

    def gathers(b):
      pltpu.async_copy(u2w.at[ebufs[b].at[0]], abufs[b], gasems[b])
      pltpu.async_copy(i2.at[ebufs[b].at[1]], bbufs[b], gbsems[b])

    def wait_gathers(b):
      pltpu.make_async_copy(u2w.at[ebufs[b].at[0]], abufs[b],
                            gasems[b]).wait()
      pltpu.make_async_copy(i2.at[ebufs[b].at[1]], bbufs[b],
                            gbsems[b]).wait()

    def wait_idx(b):
      pltpu.make_async_copy(edges.at[base], ebufs[b], isems[b]).wait()

    def compute(b, j):
      def block(k, _):
        rvec = iota + k * 16
        hv0 = bsv[pl.ds(0, 16)]
        hv1 = jnp.zeros((16,), F32)
        for cc in range(h // 2):
          cv0 = jnp.full((16,), 2 * cc, I32)
          cv1 = jnp.full((16,), 2 * cc + 1, I32)
          hv0 = hv0 + (plsc.load_gather(abufs[b], [rvec, cv0]) *
                       plsc.load_gather(bbufs[b], [rvec, cv0]))
          hv1 = hv1 + (plsc.load_gather(abufs[b], [rvec, cv1]) *
                       plsc.load_gather(bbufs[b], [rvec, cv1]))
        hbuf[pl.ds(k * 16, 16)] = hv0 + hv1
        return 0

      lax.fori_loop(0, CW // 16, block, 0)
      pltpu.sync_copy(hbuf, hout.at[pl.ds((base + j) * CW, CW)])

    pltpu.sync_copy(edges.at[base], ebufs[0])
    gathers(0)
    pltpu.sync_copy(edges.at[base + 1], ebufs[1])
    gathers(1)
    pltpu.async_copy(edges.at[base + 2], ebufs[2], isems[2])

    def quad(jj, _):
      j0 = jj * 4
      for b in range(4):
        j = j0 + b
        wait_gathers(b)
        wait_idx((b + 2) % 4)
        gathers((b + 2) % 4)
        pltpu.async_copy(edges.at[base + j + 3], ebufs[(b + 3) % 4],
                         isems[(b + 3) % 4])
        compute(b, j)
      return 0

    lax.fori_loop(0, n_chunks // 4 - 1, quad, 0)

    wait_gathers(0)
    wait_idx(2); gathers(2)
    pltpu.async_copy(edges.at[base + n_chunks - 1], ebufs[3], isems[3])
    compute(0, n_chunks - 4)
    wait_gathers(1)
    wait_idx(3); gathers(3)
    compute(1, n_chunks - 3)
    wait_gathers(2)
    compute(2, n_chunks - 2)
    wait_gathers(3)
    compute(3, n_chunks - 1)

  return pl.kernel(
      body,
      out_type=jax.ShapeDtypeStruct((b_link,), F32),
      mesh=mesh,
      scratch_types=(
          [pltpu.VMEM((2, CW), I32)] * 4
          + [pltpu.VMEM((CW, h), F32)] * 8
          + [pltpu.VMEM((CW,), F32), pltpu.VMEM((16,), F32)]
          + [pltpu.SemaphoreType.DMA] * 12
      ),
    compiler_params=pltpu.CompilerParams(use_tc_tiling_on_sc=False,
                                           needs_layout_passes=False),
  )


def _dot(a, b):
  return jnp.dot(a, b, preferred_element_type=F32,
                 precision=lax.Precision.HIGHEST)


def _pre1_body(xu, xi, w1ui, w1iu, wp1, b1iu, bp1v, yu, yi, bc1):
  yu[...] = _dot(xu[...], w1ui[...])
  wc = _dot(w1iu[...], wp1[...])
  yi[...] = _dot(xi[...], wc)
  bc1[...] = _dot(b1iu[...], wp1[...]) + bp1v[...]


def _pre2_body(user1, item1, w2ui, w2iu, wp2, b2iu, bp2v, wpostt,
               zu, zi, bc2, wsum):
  zu[...] = _dot(user1[...], w2ui[...])
  zi[...] = _dot(item1[...], _dot(w2iu[...], wp2[...]))
  bc2[...] = _dot(b2iu[...], wp2[...]) + bp2v[...]
  wsum[...] = jnp.sum(wpostt[...], axis=0, keepdims=True)


def kernel(x_user, x_item, edge_index_ui, edge_index_iu, edge_label_index,
           snap, W1_ui, b1_ui, W1_iu, b1_iu, Wp1, bp1, Ws1, bs1, qs1,
           W2_ui, b2_ui, W2_iu, b2_iu, Wp2, bp2, Ws2, bs2, qs2,
           Wpost, bpost):
  n_user, d_in = x_user.shape
  n_item = x_item.shape[0]
  h1 = W1_ui.shape[1]
  h2 = W2_ui.shape[1]
  e = edge_index_ui.shape[1]
  b_link = edge_label_index.shape[1]

  e_pad = ((e + 64 * CW - 1) // (64 * CW)) * (64 * CW)
  npd = NPAD

  xu_p = jnp.pad(x_user, ((0, npd - n_user), (0, 0)))
  xi_p = jnp.pad(x_item, ((0, npd - n_item), (0, 0)))
  fill = (npd - 240) + (jnp.arange(e_pad - e, dtype=I32) % 240)
  def pad_edges(ei):
    src = jnp.concatenate([ei[0].astype(I32), fill]).reshape(-1, 1, CW)
    dst = jnp.concatenate([ei[1].astype(I32), fill]).reshape(-1, 1, CW)
    return jnp.concatenate([src, dst], axis=1)
  eui3 = pad_edges(edge_index_ui)
  eiu3 = pad_edges(edge_index_iu)
  elab3 = jnp.concatenate(
      [edge_label_index[0].astype(I32).reshape(-1, 1, CW),
       edge_label_index[1].astype(I32).reshape(-1, 1, CW)], axis=1)

  grid = 8
  blk = npd // grid
  yu, yi, bc1 = pl.pallas_call(
      _pre1_body,
      grid=(grid,),
      in_specs=[
          pl.BlockSpec((blk, d_in), lambda i: (i, 0)),
          pl.BlockSpec((blk, d_in), lambda i: (i, 0)),
          pl.BlockSpec((d_in, h1), lambda i: (0, 0)),
          pl.BlockSpec((d_in, h1), lambda i: (0, 0)),
          pl.BlockSpec((h1, h1), lambda i: (0, 0)),
          pl.BlockSpec((1, h1), lambda i: (0, 0)),
          pl.BlockSpec((1, h1), lambda i: (0, 0)),
      ],
      out_specs=[
          pl.BlockSpec((blk, h1), lambda i: (i, 0)),
          pl.BlockSpec((blk, h1), lambda i: (i, 0)),
          pl.BlockSpec((1, h1), lambda i: (0, 0)),
      ],
      out_shape=[
          jax.ShapeDtypeStruct((npd, h1), F32),
          jax.ShapeDtypeStruct((npd, h1), F32),
          jax.ShapeDtypeStruct((1, h1), F32),
      ],
  )(xu_p, xi_p, W1_ui, W1_iu, Wp1, b1_iu.reshape(1, h1), bp1.reshape(1, h1))

  k2 = _make_scatter_kernel(h1, e_pad, layer=1)
  item1p, user1p, rdeg_i, rdeg_u = k2(yu, yi, eui3, eiu3,
                                      b1_ui, bc1.reshape(h1))

  zu, zi, bc2, wsum = pl.pallas_call(
      _pre2_body,
      grid=(grid,),
      in_specs=[
          pl.BlockSpec((blk, h1), lambda i: (i, 0)),
          pl.BlockSpec((blk, h1), lambda i: (i, 0)),
          pl.BlockSpec((h1, h2), lambda i: (0, 0)),
          pl.BlockSpec((h1, h2), lambda i: (0, 0)),
          pl.BlockSpec((h2, h2), lambda i: (0, 0)),
          pl.BlockSpec((1, h2), lambda i: (0, 0)),
          pl.BlockSpec((1, h2), lambda i: (0, 0)),
          pl.BlockSpec((2, h2), lambda i: (0, 0)),
      ],
      out_specs=[pl.BlockSpec((blk, h2), lambda i: (i, 0))] * 2
      + [pl.BlockSpec((1, h2), lambda i: (0, 0))] * 2,
      out_shape=[
          jax.ShapeDtypeStruct((npd, h2), F32),
          jax.ShapeDtypeStruct((npd, h2), F32),
          jax.ShapeDtypeStruct((1, h2), F32),
          jax.ShapeDtypeStruct((1, h2), F32),
      ],
  )(user1p, item1p, W2_ui, W2_iu, Wp2, b2_iu.reshape(1, h2),
    bp2.reshape(1, h2), Wpost.T)

  k4 = _make_scatter_kernel(h2, e_pad, layer=2)
  item2p, user2p, u2wp = k4(zu, zi, eui3, eiu3, b2_ui, bc2.reshape(h2),
                            rdeg_i, rdeg_u, wsum.reshape(h2))

  bsum = jnp.broadcast_to(jnp.sum(bpost), (16,)).astype(F32)
  k6 = _make_head_kernel(b_link, h2)
  h = k6(u2wp, item2p, elab3, bsum)

  return (h, user1p[:n_user], item1p[:n_item],
          user2p[:n_user], item2p[:n_item])


# speedup vs baseline: 1.2854x; 1.1226x over previous
import functools

import jax
import jax.numpy as jnp
from jax import lax
from jax.experimental import pallas as pl
from jax.experimental.pallas import tpu as pltpu
from jax.experimental.pallas import tpu_sc as plsc

F32 = jnp.float32
I32 = jnp.int32

NPAD = 10240
CW = 256
RW = 128
ROWS_PER_SUB = NPAD // 16


def _zero_rows(rows, width):
  z = jnp.zeros((16,), F32)

  def body(r, _):
    for h in range(width // 16):
      rows[r, pl.ds(h * 16, 16)] = z
    return 0

  lax.fori_loop(0, 128, body, 0)


def _zero_vec(buf, n):
  z = jnp.zeros((16,), F32)
  for k in range(n // 16):
    buf[pl.ds(k * 16, 16)] = z


def _make_scatter_kernel(h, e_pad, layer):
  n_per_sub = e_pad // 16
  cps = n_per_sub // CW
  assert cps * CW == n_per_sub and cps % 4 == 0 and cps >= 8
  with_deg = layer == 1

  mesh = plsc.VectorSubcoreMesh(core_axis_name="c", subcore_axis_name="s",
                                num_cores=2, num_subcores=16)
  out_type = [
      jax.ShapeDtypeStruct((NPAD, h), F32),
      jax.ShapeDtypeStruct((NPAD, h), F32),
  ]
  nbuf = 4
  scratch = (
      [pltpu.VMEM_SHARED((NPAD, h), F32)]
      + ([pltpu.VMEM_SHARED((NPAD,), F32)] if with_deg else [])
      + [pltpu.VMEM((2, CW), I32)] * nbuf
      + [pltpu.VMEM((CW, h), F32)] * nbuf
      + [pltpu.VMEM((CW,), F32)]
      + [pltpu.VMEM((CW,), F32)]
      + [pltpu.VMEM((h,), F32)]
      + [pltpu.VMEM((h,), F32)]
      + [pltpu.SemaphoreType.DMA] * (2 * nbuf + 1)
  )
  if with_deg:
    out_type += [
        jax.ShapeDtypeStruct((NPAD,), F32),
        jax.ShapeDtypeStruct((NPAD,), F32),
    ]
  else:
    out_type += [jax.ShapeDtypeStruct((NPAD, h), F32)]

  def body(*refs):
    if with_deg:
      (ta, tb, ea, eb, biasa, biasb, oa, ob, dega, degb,
       acc, dacc) = refs[:12]
      rest = refs[12:]
      rdega = rdegb = wsum = u2w = None
    else:
      (ta, tb, ea, eb, biasa, biasb, rdega, rdegb, wsum,
       oa, ob, u2w, acc) = refs[:13]
      rest = refs[13:]
      dacc = dega = degb = None
    ebufs = rest[:4]
    rbufs = rest[4:8]
    fbuf = rest[8]
    dbuf = rest[9]
    bbuf = rest[10]
    wbuf = rest[11]
    gsems = rest[12:16]
    isems = rest[16:20]
    dsem = rest[20]
    r0buf = rbufs[0]
    c = lax.axis_index("c")
    s = lax.axis_index("s")
    r0 = s * ROWS_PER_SUB

    _zero_rows(r0buf, h)
    for k in range(ROWS_PER_SUB // RW):
      pltpu.sync_copy(r0buf.at[pl.ds(0, RW)], acc.at[pl.ds(r0 + k * RW, RW)])
    if with_deg:
      _zero_vec(fbuf, CW)
      for k in range(ROWS_PER_SUB // RW):
        pltpu.sync_copy(fbuf.at[pl.ds(0, RW)],
                        dacc.at[pl.ds(r0 + k * RW, RW)])
      one = jnp.ones((16,), F32)
      for k in range(CW // 16):
        fbuf[pl.ds(k * 16, 16)] = one

    plsc.subcore_barrier()

    def process(table, edges):
      base = s * cps

      def gather(b):
        pltpu.async_copy(table.at[ebufs[b].at[0]], rbufs[b], gsems[b])

      def wait_gather(b):
        pltpu.make_async_copy(table.at[ebufs[b].at[0]], rbufs[b],
                              gsems[b]).wait()

      def wait_idx(b):
        pltpu.make_async_copy(edges.at[base], ebufs[b], isems[b]).wait()

      def scatter(b):
        if with_deg:
          pltpu.async_copy(fbuf, dacc.at[ebufs[b].at[1]], dsem, add=True)
        pltpu.sync_copy(rbufs[b], acc.at[ebufs[b].at[1]], add=True)
        if with_deg:
          pltpu.make_async_copy(fbuf, dacc.at[ebufs[b].at[1]], dsem).wait()

      pltpu.sync_copy(edges.at[base], ebufs[0])
      gather(0)
      pltpu.sync_copy(edges.at[base + 1], ebufs[1])
      gather(1)
      pltpu.async_copy(edges.at[base + 2], ebufs[2], isems[2])

      def quad(jj, _):
        j0 = jj * 4
        for b in range(4):
          j = j0 + b
          wait_gather(b)
          scatter(b)
          wait_idx((b + 2) % 4)
          gather((b + 2) % 4)
          pltpu.async_copy(edges.at[base + j + 3], ebufs[(b + 3) % 4],
                           isems[(b + 3) % 4])
        return 0

      lax.fori_loop(0, cps // 4 - 1, quad, 0)

      wait_gather(0); scatter(0)
      wait_idx(2); gather(2)
      pltpu.async_copy(edges.at[base + cps - 1], ebufs[3], isems[3])
      wait_gather(1); scatter(1)
      wait_idx(3); gather(3)
      wait_gather(2); scatter(2)
      wait_gather(3); scatter(3)

    @pl.when(c == 0)
    def _():
      process(ta, ea)

    @pl.when(c == 1)
    def _():
      process(tb, eb)

    plsc.subcore_barrier()

    def writeout(out, bias, deg_out, rdeg_in, with_u2w):
      pltpu.sync_copy(bias, bbuf)
      if with_u2w:
        pltpu.sync_copy(wsum, wbuf)
      bias_ch = [bbuf[pl.ds(cc * 16, 16)] for cc in range(h // 16)]
      w_ch = ([wbuf[pl.ds(cc * 16, 16)] for cc in range(h // 16)]
              if with_u2w else None)
      for k in range(ROWS_PER_SUB // RW):
        blk = r0 + k * RW
        pltpu.sync_copy(acc.at[pl.ds(blk, RW)], r0buf.at[pl.ds(0, RW)])
        if with_deg:
          pltpu.sync_copy(dacc.at[pl.ds(blk, RW)], dbuf.at[pl.ds(0, RW)])
          for kk in range(RW // 16):
            d = dbuf[pl.ds(kk * 16, 16)]
            dbuf[pl.ds(kk * 16, 16)] = 1.0 / jnp.maximum(d, 1.0)
          pltpu.sync_copy(dbuf.at[pl.ds(0, RW)], deg_out.at[pl.ds(blk, RW)])
        else:
          pltpu.sync_copy(rdeg_in.at[pl.ds(blk, RW)], dbuf.at[pl.ds(0, RW)])

        def rowgrp(g, _):
          rv16 = dbuf[pl.ds(g * 16, 16)]
          for i in range(16):
            r = g * 16 + i
            rv = rv16[i]
            for cc in range(h // 16):
              x = r0buf[r, pl.ds(cc * 16, 16)]
              y = x * rv + bias_ch[cc]
              r0buf[r, pl.ds(cc * 16, 16)] = y
              if with_u2w:
                rbufs[1][r, pl.ds(cc * 16, 16)] = y * w_ch[cc]
          return 0

        lax.fori_loop(0, RW // 16, rowgrp, 0)
        pltpu.sync_copy(r0buf.at[pl.ds(0, RW)], out.at[pl.ds(blk, RW)])
        if with_u2w:
          pltpu.sync_copy(rbufs[1].at[pl.ds(0, RW)], u2w.at[pl.ds(blk, RW)])

    @pl.when(c == 0)
    def _():
      writeout(oa, biasa, dega, rdega, False)

    @pl.when(c == 1)
    def _():
      writeout(ob, biasb, degb, rdegb, not with_deg)

  return pl.kernel(body, out_type=out_type, mesh=mesh, scratch_types=scratch,
                   compiler_params=pltpu.CompilerParams(
                       use_tc_tiling_on_sc=False))


def _make_head_kernel(b_link, h):
  n_per_w = b_link // 32
  n_chunks = n_per_w // CW
  assert n_chunks * CW == n_per_w and n_chunks % 4 == 0 and n_chunks >= 8

  mesh = plsc.VectorSubcoreMesh(core_axis_name="c", subcore_axis_name="s",
                                num_cores=2, num_subcores=16)

  def body(*refs):
    (u2w, i2, edges, bsum, hout) = refs[:5]
    rest = refs[5:]
    ebufs = rest[:4]
    abufs = rest[4:8]
    bbufs = rest[8:12]
    hbuf = rest[12]
    bsv = rest[13]
    gasems = rest[14:18]
    gbsems = rest[18:22]
    isems = rest[22:26]
    c = lax.axis_index("c")
    s = lax.axis_index("s")
    wid = s * 2 + c
    base = wid * n_chunks
    pltpu.sync_copy(bsum, bsv)
    iota = lax.iota(I32, 16)

    def gathers(b):
      pltpu.async_copy(u2w.at[ebufs[b].at[0]], abufs[b], gasems[b])
      pltpu.async_copy(i2.at[ebufs[b].at[1]], bbufs[b], gbsems[b])

    def wait_gathers(b):
      pltpu.make_async_copy(u2w.at[ebufs[b].at[0]], abufs[b],
                            gasems[b]).wait()
      pltpu.make_async_copy(i2.at[ebufs[b].at[1]], bbufs[b],
                            gbsems[b]).wait()

    def wait_idx(b):
      pltpu.make_async_copy(edges.at[base], ebufs[b], isems[b]).wait()

    def compute(b, j):
      def block(k, _):
        rvec = iota + k * 16
        hv0 = bsv[pl.ds(0, 16)]
        hv1 = jnp.zeros((16,), F32)
        for cc in range(h // 2):
          cv0 = jnp.full((16,), 2 * cc, I32)
          cv1 = jnp.full((16,), 2 * cc + 1, I32)
          hv0 = hv0 + (plsc.load_gather(abufs[b], [rvec, cv0]) *
                       plsc.load_gather(bbufs[b], [rvec, cv0]))
          hv1 = hv1 + (plsc.load_gather(abufs[b], [rvec, cv1]) *
                       plsc.load_gather(bbufs[b], [rvec, cv1]))
        hbuf[pl.ds(k * 16, 16)] = hv0 + hv1
        return 0

      lax.fori_loop(0, CW // 16, block, 0)
      pltpu.sync_copy(hbuf, hout.at[pl.ds((base + j) * CW, CW)])

    pltpu.sync_copy(edges.at[base], ebufs[0])
    gathers(0)
    pltpu.sync_copy(edges.at[base + 1], ebufs[1])
    gathers(1)
    pltpu.async_copy(edges.at[base + 2], ebufs[2], isems[2])

    def quad(jj, _):
      j0 = jj * 4
      for b in range(4):
        j = j0 + b
        wait_gathers(b)
        wait_idx((b + 2) % 4)
        gathers((b + 2) % 4)
        pltpu.async_copy(edges.at[base + j + 3], ebufs[(b + 3) % 4],
                         isems[(b + 3) % 4])
        compute(b, j)
      return 0

    lax.fori_loop(0, n_chunks // 4 - 1, quad, 0)

    wait_gathers(0)
    wait_idx(2); gathers(2)
    pltpu.async_copy(edges.at[base + n_chunks - 1], ebufs[3], isems[3])
    compute(0, n_chunks - 4)
    wait_gathers(1)
    wait_idx(3); gathers(3)
    compute(1, n_chunks - 3)
    wait_gathers(2)
    compute(2, n_chunks - 2)
    wait_gathers(3)
    compute(3, n_chunks - 1)

  return pl.kernel(
      body,
      out_type=jax.ShapeDtypeStruct((b_link,), F32),
      mesh=mesh,
      scratch_types=(
          [pltpu.VMEM((2, CW), I32)] * 4
          + [pltpu.VMEM((CW, h), F32)] * 8
          + [pltpu.VMEM((CW,), F32), pltpu.VMEM((16,), F32)]
          + [pltpu.SemaphoreType.DMA] * 12
      ),
    compiler_params=pltpu.CompilerParams(use_tc_tiling_on_sc=False,
                                           needs_layout_passes=False),
  )


def _dot(a, b):
  return jnp.dot(a, b, preferred_element_type=F32,
                 precision=lax.Precision.HIGHEST)


def _pre1_body(xu, xi, w1ui, w1iu, wp1, b1iu, bp1v, yu, yi, bc1):
  yu[...] = _dot(xu[...], w1ui[...])
  wc = _dot(w1iu[...], wp1[...])
  yi[...] = _dot(xi[...], wc)
  bc1[...] = _dot(b1iu[...], wp1[...]) + bp1v[...]


def _pre2_body(user1, item1, w2ui, w2iu, wp2, b2iu, bp2v, wpostt,
               zu, zi, bc2, wsum):
  zu[...] = _dot(user1[...], w2ui[...])
  zi[...] = _dot(item1[...], _dot(w2iu[...], wp2[...]))
  bc2[...] = _dot(b2iu[...], wp2[...]) + bp2v[...]
  wsum[...] = jnp.sum(wpostt[...], axis=0, keepdims=True)


def kernel(x_user, x_item, edge_index_ui, edge_index_iu, edge_label_index,
           snap, W1_ui, b1_ui, W1_iu, b1_iu, Wp1, bp1, Ws1, bs1, qs1,
           W2_ui, b2_ui, W2_iu, b2_iu, Wp2, bp2, Ws2, bs2, qs2,
           Wpost, bpost):
  n_user, d_in = x_user.shape
  n_item = x_item.shape[0]
  h1 = W1_ui.shape[1]
  h2 = W2_ui.shape[1]
  e = edge_index_ui.shape[1]
  b_link = edge_label_index.shape[1]

  e_pad = ((e + 64 * CW - 1) // (64 * CW)) * (64 * CW)
  npd = NPAD

  xu_p = jnp.pad(x_user, ((0, npd - n_user), (0, 0)))
  xi_p = jnp.pad(x_item, ((0, npd - n_item), (0, 0)))
  fill = (npd - 240) + (jnp.arange(e_pad - e, dtype=I32) % 240)
  def pad_edges(ei):
    src = jnp.concatenate([ei[0].astype(I32), fill]).reshape(-1, 1, CW)
    dst = jnp.concatenate([ei[1].astype(I32), fill]).reshape(-1, 1, CW)
    return jnp.concatenate([src, dst], axis=1)
  eui3 = pad_edges(edge_index_ui)
  eiu3 = pad_edges(edge_index_iu)
  elab3 = jnp.concatenate(
      [edge_label_index[0].astype(I32).reshape(-1, 1, CW),
       edge_label_index[1].astype(I32).reshape(-1, 1, CW)], axis=1)

  grid = 8
  blk = npd // grid
  yu, yi, bc1 = pl.pallas_call(
      _pre1_body,
      grid=(grid,),
      in_specs=[
          pl.BlockSpec((blk, d_in), lambda i: (i, 0)),
          pl.BlockSpec((blk, d_in), lambda i: (i, 0)),
          pl.BlockSpec((d_in, h1), lambda i: (0, 0)),
          pl.BlockSpec((d_in, h1), lambda i: (0, 0)),
          pl.BlockSpec((h1, h1), lambda i: (0, 0)),
          pl.BlockSpec((1, h1), lambda i: (0, 0)),
          pl.BlockSpec((1, h1), lambda i: (0, 0)),
      ],
      out_specs=[
          pl.BlockSpec((blk, h1), lambda i: (i, 0)),
          pl.BlockSpec((blk, h1), lambda i: (i, 0)),
          pl.BlockSpec((1, h1), lambda i: (0, 0)),
      ],
      out_shape=[
          jax.ShapeDtypeStruct((npd, h1), F32),
          jax.ShapeDtypeStruct((npd, h1), F32),
          jax.ShapeDtypeStruct((1, h1), F32),
      ],
  )(xu_p, xi_p, W1_ui, W1_iu, Wp1, b1_iu.reshape(1, h1), bp1.reshape(1, h1))

  k2 = _make_scatter_kernel(h1, e_pad, layer=1)
  item1p, user1p, rdeg_i, rdeg_u = k2(yu, yi, eui3, eiu3,
                                      b1_ui, bc1.reshape(h1))

  zu, zi, bc2, wsum = pl.pallas_call(
      _pre2_body,
      grid=(grid,),
      in_specs=[
          pl.BlockSpec((blk, h1), lambda i: (i, 0)),
          pl.BlockSpec((blk, h1), lambda i: (i, 0)),
          pl.BlockSpec((h1, h2), lambda i: (0, 0)),
          pl.BlockSpec((h1, h2), lambda i: (0, 0)),
          pl.BlockSpec((h2, h2), lambda i: (0, 0)),
          pl.BlockSpec((1, h2), lambda i: (0, 0)),
          pl.BlockSpec((1, h2), lambda i: (0, 0)),
          pl.BlockSpec((2, h2), lambda i: (0, 0)),
      ],
      out_specs=[pl.BlockSpec((blk, h2), lambda i: (i, 0))] * 2
      + [pl.BlockSpec((1, h2), lambda i: (0, 0))] * 2,
      out_shape=[
          jax.ShapeDtypeStruct((npd, h2), F32),
          jax.ShapeDtypeStruct((npd, h2), F32),
          jax.ShapeDtypeStruct((1, h2), F32),
          jax.ShapeDtypeStruct((1, h2), F32),
      ],
  )(user1p, item1p, W2_ui, W2_iu, Wp2, b2_iu.reshape(1, h2),
    bp2.reshape(1, h2), Wpost.T)

  k4 = _make_scatter_kernel(h2, e_pad, layer=2)
  item2p, user2p, u2wp = k4(zu, zi, eui3, eiu3, b2_ui, bc2.reshape(h2),
                            rdeg_i, rdeg_u, wsum.reshape(h2))

  bsum = jnp.broadcast_to(jnp.sum(bpost), (16,)).astype(F32)
  k6 = _make_head_kernel(b_link, h2)
  h = k6(u2wp, item2p, elab3, bsum)

  return (h, user1p[:n_user], item1p[:n_item],
          user2p[:n_user], item2p[:n_item])


# speedup vs baseline: 1.2870x; 1.0012x over previous
import jax
import jax.numpy as jnp
from jax import lax
from jax.experimental import pallas as pl
from jax.experimental.pallas import tpu as pltpu
from jax.experimental.pallas import tpu_sc as plsc

F32 = jnp.float32
I32 = jnp.int32

NPAD = 10240
CW = 256
RW = 128
ROWS_PER_SUB = NPAD // 16


def _zero_rows(rows, width):
  z = jnp.zeros((16,), F32)

  def body(r, _):
    for h in range(width // 16):
      rows[r, pl.ds(h * 16, 16)] = z
    return 0

  lax.fori_loop(0, 128, body, 0)


def _zero_vec(buf, n):
  z = jnp.zeros((16,), F32)
  for k in range(n // 16):
    buf[pl.ds(k * 16, 16)] = z


def _make_scatter_kernel(h, e_pad, layer):
  n_per_sub = e_pad // 16
  cps = n_per_sub // CW
  assert cps * CW == n_per_sub and cps % 4 == 0 and cps >= 8
  with_deg = layer == 1

  mesh = plsc.VectorSubcoreMesh(core_axis_name="c", subcore_axis_name="s",
                                num_cores=2, num_subcores=16)
  out_type = [
      jax.ShapeDtypeStruct((NPAD, h), F32),
      jax.ShapeDtypeStruct((NPAD, h), F32),
  ]
  nbuf = 4
  scratch = (
      [pltpu.VMEM_SHARED((NPAD, h), F32)]
      + ([pltpu.VMEM_SHARED((NPAD,), F32)] if with_deg else [])
      + [pltpu.VMEM((2, CW), I32)] * nbuf
      + [pltpu.VMEM((CW, h), F32)] * nbuf
      + [pltpu.VMEM((CW,), F32)]
      + [pltpu.VMEM((CW,), F32)]
      + [pltpu.VMEM((h,), F32)]
      + [pltpu.VMEM((h,), F32)]
      + [pltpu.SemaphoreType.DMA] * (2 * nbuf + 1)
  )
  if with_deg:
    out_type += [
        jax.ShapeDtypeStruct((NPAD,), F32),
        jax.ShapeDtypeStruct((NPAD,), F32),
    ]
  else:
    out_type += [jax.ShapeDtypeStruct((NPAD, h), F32)]

  def body(*refs):
    if with_deg:
      (ta, tb, ea, eb, biasa, biasb, oa, ob, dega, degb,
       acc, dacc) = refs[:12]
      rest = refs[12:]
      rdega = rdegb = wsum = u2w = None
    else:
      (ta, tb, ea, eb, biasa, biasb, rdega, rdegb, wsum,
       oa, ob, u2w, acc) = refs[:13]
      rest = refs[13:]
      dacc = dega = degb = None
    ebufs = rest[:4]
    rbufs = rest[4:8]
    fbuf = rest[8]
    dbuf = rest[9]
    bbuf = rest[10]
    wbuf = rest[11]
    gsems = rest[12:16]
    isems = rest[16:20]
    dsem = rest[20]
    r0buf = rbufs[0]
    c = lax.axis_index("c")
    s = lax.axis_index("s")
    r0 = s * ROWS_PER_SUB

    _zero_rows(r0buf, h)
    for k in range(ROWS_PER_SUB // RW):
      pltpu.sync_copy(r0buf.at[pl.ds(0, RW)], acc.at[pl.ds(r0 + k * RW, RW)])
    if with_deg:
      _zero_vec(fbuf, CW)
      for k in range(ROWS_PER_SUB // RW):
        pltpu.sync_copy(fbuf.at[pl.ds(0, RW)],
                        dacc.at[pl.ds(r0 + k * RW, RW)])
      one = jnp.ones((16,), F32)
      for k in range(CW // 16):
        fbuf[pl.ds(k * 16, 16)] = one

    plsc.subcore_barrier()

    def process(table, edges):
      base = s * cps

      def gather(b):
        pltpu.async_copy(table.at[ebufs[b].at[0]], rbufs[b], gsems[b])

      def wait_gather(b):
        pltpu.make_async_copy(table.at[ebufs[b].at[0]], rbufs[b],
                              gsems[b]).wait()

      def wait_idx(b):
        pltpu.make_async_copy(edges.at[base], ebufs[b], isems[b]).wait()

      def scatter(b):
        if with_deg:
          pltpu.async_copy(fbuf, dacc.at[ebufs[b].at[1]], dsem, add=True)
        pltpu.sync_copy(rbufs[b], acc.at[ebufs[b].at[1]], add=True)
        if with_deg:
          pltpu.make_async_copy(fbuf, dacc.at[ebufs[b].at[1]], dsem).wait()

      pltpu.sync_copy(edges.at[base], ebufs[0])
      gather(0)
      pltpu.sync_copy(edges.at[base + 1], ebufs[1])
      gather(1)
      pltpu.async_copy(edges.at[base + 2], ebufs[2], isems[2])

      def quad(jj, _):
        j0 = jj * 4
        for b in range(4):
          j = j0 + b
          wait_gather(b)
          scatter(b)
          wait_idx((b + 2) % 4)
          gather((b + 2) % 4)
          pltpu.async_copy(edges.at[base + j + 3], ebufs[(b + 3) % 4],
                           isems[(b + 3) % 4])
        return 0

      lax.fori_loop(0, cps // 4 - 1, quad, 0)

      wait_gather(0); scatter(0)
      wait_idx(2); gather(2)
      pltpu.async_copy(edges.at[base + cps - 1], ebufs[3], isems[3])
      wait_gather(1); scatter(1)
      wait_idx(3); gather(3)
      wait_gather(2); scatter(2)
      wait_gather(3); scatter(3)

    @pl.when(c == 0)
    def _():
      process(ta, ea)

    @pl.when(c == 1)
    def _():
      process(tb, eb)

    plsc.subcore_barrier()

    def writeout(out, bias, deg_out, rdeg_in, with_u2w):
      pltpu.sync_copy(bias, bbuf)
      if with_u2w:
        pltpu.sync_copy(wsum, wbuf)
      bias_ch = [bbuf[pl.ds(cc * 16, 16)] for cc in range(h // 16)]
      w_ch = ([wbuf[pl.ds(cc * 16, 16)] for cc in range(h // 16)]
              if with_u2w else None)
      for k in range(ROWS_PER_SUB // RW):
        blk = r0 + k * RW
        pltpu.sync_copy(acc.at[pl.ds(blk, RW)], r0buf.at[pl.ds(0, RW)])
        if with_deg:
          pltpu.sync_copy(dacc.at[pl.ds(blk, RW)], dbuf.at[pl.ds(0, RW)])
          for kk in range(RW // 16):
            d = dbuf[pl.ds(kk * 16, 16)]
            dbuf[pl.ds(kk * 16, 16)] = 1.0 / jnp.maximum(d, 1.0)
          pltpu.sync_copy(dbuf.at[pl.ds(0, RW)], deg_out.at[pl.ds(blk, RW)])
        else:
          pltpu.sync_copy(rdeg_in.at[pl.ds(blk, RW)], dbuf.at[pl.ds(0, RW)])

        def rowgrp(g, _):
          rv16 = dbuf[pl.ds(g * 16, 16)]
          for i in range(16):
            r = g * 16 + i
            rv = rv16[i]
            for cc in range(h // 16):
              x = r0buf[r, pl.ds(cc * 16, 16)]
              y = x * rv + bias_ch[cc]
              r0buf[r, pl.ds(cc * 16, 16)] = y
              if with_u2w:
                rbufs[1][r, pl.ds(cc * 16, 16)] = y * w_ch[cc]
          return 0

        lax.fori_loop(0, RW // 16, rowgrp, 0)
        pltpu.sync_copy(r0buf.at[pl.ds(0, RW)], out.at[pl.ds(blk, RW)])
        if with_u2w:
          pltpu.sync_copy(rbufs[1].at[pl.ds(0, RW)], u2w.at[pl.ds(blk, RW)])

    @pl.when(c == 0)
    def _():
      writeout(oa, biasa, dega, rdega, False)

    @pl.when(c == 1)
    def _():
      writeout(ob, biasb, degb, rdegb, not with_deg)

  return pl.kernel(body, out_type=out_type, mesh=mesh, scratch_types=scratch,
                   compiler_params=pltpu.CompilerParams(
                       use_tc_tiling_on_sc=False))


def _make_head_kernel(b_link, h):
  n_per_w = b_link // 32
  n_chunks = n_per_w // CW
  assert n_chunks * CW == n_per_w and n_chunks % 4 == 0 and n_chunks >= 8

  mesh = plsc.VectorSubcoreMesh(core_axis_name="c", subcore_axis_name="s",
                                num_cores=2, num_subcores=16)

  def body(*refs):
    (u2w, i2, edges, bsum, hout) = refs[:5]
    rest = refs[5:]
    ebufs = rest[:4]
    abufs = rest[4:8]
    bbufs = rest[8:12]
    hbuf = rest[12]
    bsv = rest[13]
    gasems = rest[14:18]
    gbsems = rest[18:22]
    isems = rest[22:26]
    c = lax.axis_index("c")
    s = lax.axis_index("s")
    wid = s * 2 + c
    base = wid * n_chunks
    pltpu.sync_copy(bsum, bsv)
    iota = lax.iota(I32, 16)

    def gathers(b):
      pltpu.async_copy(u2w.at[ebufs[b].at[0]], abufs[b], gasems[b])
      pltpu.async_copy(i2.at[ebufs[b].at[1]], bbufs[b], gbsems[b])

    def wait_gathers(b):
      pltpu.make_async_copy(u2w.at[ebufs[b].at[0]], abufs[b],
                            gasems[b]).wait()
      pltpu.make_async_copy(i2.at[ebufs[b].at[1]], bbufs[b],
                            gbsems[b]).wait()

    def wait_idx(b):
      pltpu.make_async_copy(edges.at[base], ebufs[b], isems[b]).wait()

    def compute(b, j):
      def block(k, _):
        rvec = iota + k * 16
        hv0 = bsv[pl.ds(0, 16)]
        hv1 = jnp.zeros((16,), F32)
        for cc in range(h // 2):
          cv0 = jnp.full((16,), 2 * cc, I32)
          cv1 = jnp.full((16,), 2 * cc + 1, I32)
          hv0 = hv0 + (plsc.load_gather(abufs[b], [rvec, cv0]) *
                       plsc.load_gather(bbufs[b], [rvec, cv0]))
          hv1 = hv1 + (plsc.load_gather(abufs[b], [rvec, cv1]) *
                       plsc.load_gather(bbufs[b], [rvec, cv1]))
        hbuf[pl.ds(k * 16, 16)] = hv0 + hv1
        return 0

      lax.fori_loop(0, CW // 16, block, 0)
      pltpu.sync_copy(hbuf, hout.at[pl.ds((base + j) * CW, CW)])

    pltpu.sync_copy(edges.at[base], ebufs[0])
    gathers(0)
    pltpu.sync_copy(edges.at[base + 1], ebufs[1])
    gathers(1)
    pltpu.async_copy(edges.at[base + 2], ebufs[2], isems[2])

    def quad(jj, _):
      j0 = jj * 4
      for b in range(4):
        j = j0 + b
        wait_gathers(b)
        wait_idx((b + 2) % 4)
        gathers((b + 2) % 4)
        pltpu.async_copy(edges.at[base + j + 3], ebufs[(b + 3) % 4],
                         isems[(b + 3) % 4])
        compute(b, j)
      return 0

    lax.fori_loop(0, n_chunks // 4 - 1, quad, 0)

    wait_gathers(0)
    wait_idx(2); gathers(2)
    pltpu.async_copy(edges.at[base + n_chunks - 1], ebufs[3], isems[3])
    compute(0, n_chunks - 4)
    wait_gathers(1)
    wait_idx(3); gathers(3)
    compute(1, n_chunks - 3)
    wait_gathers(2)
    compute(2, n_chunks - 2)
    wait_gathers(3)
    compute(3, n_chunks - 1)

  return pl.kernel(
      body,
      out_type=jax.ShapeDtypeStruct((b_link,), F32),
      mesh=mesh,
      scratch_types=(
          [pltpu.VMEM((2, CW), I32)] * 4
          + [pltpu.VMEM((CW, h), F32)] * 8
          + [pltpu.VMEM((CW,), F32), pltpu.VMEM((16,), F32)]
          + [pltpu.SemaphoreType.DMA] * 12
      ),
    compiler_params=pltpu.CompilerParams(use_tc_tiling_on_sc=False,
                                           needs_layout_passes=False),
  )


def _dot(a, b):
  return jnp.dot(a, b, preferred_element_type=F32,
                 precision=lax.Precision.HIGHEST)


def _pre1_body(xu, xi, w1ui, w1iu, wp1, b1iu, bp1v, yu, yi, bc1):
  yu[...] = _dot(xu[...], w1ui[...])
  wc = _dot(w1iu[...], wp1[...])
  yi[...] = _dot(xi[...], wc)
  bc1[...] = _dot(b1iu[...], wp1[...]) + bp1v[...]


def _pre2_body(user1, item1, w2ui, w2iu, wp2, b2iu, bp2v, wpostt,
               zu, zi, bc2, wsum):
  zu[...] = _dot(user1[...], w2ui[...])
  zi[...] = _dot(item1[...], _dot(w2iu[...], wp2[...]))
  bc2[...] = _dot(b2iu[...], wp2[...]) + bp2v[...]
  wsum[...] = jnp.sum(wpostt[...], axis=0, keepdims=True)


def kernel(x_user, x_item, edge_index_ui, edge_index_iu, edge_label_index,
           snap, W1_ui, b1_ui, W1_iu, b1_iu, Wp1, bp1, Ws1, bs1, qs1,
           W2_ui, b2_ui, W2_iu, b2_iu, Wp2, bp2, Ws2, bs2, qs2,
           Wpost, bpost):
  n_user, d_in = x_user.shape
  n_item = x_item.shape[0]
  h1 = W1_ui.shape[1]
  h2 = W2_ui.shape[1]
  e = edge_index_ui.shape[1]
  b_link = edge_label_index.shape[1]

  e_pad = ((e + 64 * CW - 1) // (64 * CW)) * (64 * CW)
  npd = NPAD

  xu_p = jnp.pad(x_user, ((0, npd - n_user), (0, 0)))
  xi_p = jnp.pad(x_item, ((0, npd - n_item), (0, 0)))
  fill = (npd - 240) + (jnp.arange(e_pad - e, dtype=I32) % 240)
  def pad_edges(ei):
    src = jnp.concatenate([ei[0].astype(I32), fill]).reshape(-1, 1, CW)
    dst = jnp.concatenate([ei[1].astype(I32), fill]).reshape(-1, 1, CW)
    return jnp.concatenate([src, dst], axis=1)
  eui3 = pad_edges(edge_index_ui)
  eiu3 = pad_edges(edge_index_iu)
  elab3 = jnp.concatenate(
      [edge_label_index[0].astype(I32).reshape(-1, 1, CW),
       edge_label_index[1].astype(I32).reshape(-1, 1, CW)], axis=1)

  grid = 8
  blk = npd // grid
  yu, yi, bc1 = pl.pallas_call(
      _pre1_body,
      grid=(grid,),
      in_specs=[
          pl.BlockSpec((blk, d_in), lambda i: (i, 0)),
          pl.BlockSpec((blk, d_in), lambda i: (i, 0)),
          pl.BlockSpec((d_in, h1), lambda i: (0, 0)),
          pl.BlockSpec((d_in, h1), lambda i: (0, 0)),
          pl.BlockSpec((h1, h1), lambda i: (0, 0)),
          pl.BlockSpec((1, h1), lambda i: (0, 0)),
          pl.BlockSpec((1, h1), lambda i: (0, 0)),
      ],
      out_specs=[
          pl.BlockSpec((blk, h1), lambda i: (i, 0)),
          pl.BlockSpec((blk, h1), lambda i: (i, 0)),
          pl.BlockSpec((1, h1), lambda i: (0, 0)),
      ],
      out_shape=[
          jax.ShapeDtypeStruct((npd, h1), F32),
          jax.ShapeDtypeStruct((npd, h1), F32),
          jax.ShapeDtypeStruct((1, h1), F32),
      ],
  )(xu_p, xi_p, W1_ui, W1_iu, Wp1, b1_iu.reshape(1, h1), bp1.reshape(1, h1))

  k2 = _make_scatter_kernel(h1, e_pad, layer=1)
  item1p, user1p, rdeg_i, rdeg_u = k2(yu, yi, eui3, eiu3,
                                      b1_ui, bc1.reshape(h1))

  zu, zi, bc2, wsum = pl.pallas_call(
      _pre2_body,
      grid=(grid,),
      in_specs=[
          pl.BlockSpec((blk, h1), lambda i: (i, 0)),
          pl.BlockSpec((blk, h1), lambda i: (i, 0)),
          pl.BlockSpec((h1, h2), lambda i: (0, 0)),
          pl.BlockSpec((h1, h2), lambda i: (0, 0)),
          pl.BlockSpec((h2, h2), lambda i: (0, 0)),
          pl.BlockSpec((1, h2), lambda i: (0, 0)),
          pl.BlockSpec((1, h2), lambda i: (0, 0)),
          pl.BlockSpec((2, h2), lambda i: (0, 0)),
      ],
      out_specs=[pl.BlockSpec((blk, h2), lambda i: (i, 0))] * 2
      + [pl.BlockSpec((1, h2), lambda i: (0, 0))] * 2,
      out_shape=[
          jax.ShapeDtypeStruct((npd, h2), F32),
          jax.ShapeDtypeStruct((npd, h2), F32),
          jax.ShapeDtypeStruct((1, h2), F32),
          jax.ShapeDtypeStruct((1, h2), F32),
      ],
  )(user1p, item1p, W2_ui, W2_iu, Wp2, b2_iu.reshape(1, h2),
    bp2.reshape(1, h2), Wpost.T)

  k4 = _make_scatter_kernel(h2, e_pad, layer=2)
  item2p, user2p, u2wp = k4(zu, zi, eui3, eiu3, b2_ui, bc2.reshape(h2),
                            rdeg_i, rdeg_u, wsum.reshape(h2))

  bsum = jnp.broadcast_to(jnp.sum(bpost), (16,)).astype(F32)
  k6 = _make_head_kernel(b_link, h2)
  h = k6(u2wp, item2p, elab3, bsum)

  return (h, user1p[:n_user], item1p[:n_item],
          user2p[:n_user], item2p[:n_item])
